# Initial kernel scaffold; baseline (speedup 1.0000x reference)
import jax
import jax.numpy as jnp
from jax.experimental import pallas as pl


def kernel(z_e, latent_basis, W):
    raise NotImplementedError("write your pallas kernel here")


# speedup vs baseline: 1.5980x; 1.5980x over previous
import functools
import math

import jax
import jax.numpy as jnp
from jax import lax
from jax.experimental import pallas as pl
from jax.experimental.pallas import tpu as pltpu
from jax.experimental.pallas import tpu_sc as plsc

K = 8192
D = 256
NTOK = 32 * 1024
BM = 256
GRID_M = NTOK // BM
COMMIT = 0.25

NC, NS = 2, 16
NW = NC * NS
TOK_PER_W = NTOK // NW
CHUNK = 128
NCHUNK = TOK_PER_W // CHUNK


def _codebook_body(lb_ref, w_ref, cb_ref, cbn_ref):
    lb = lb_ref[...]
    w = w_ref[...]
    cb = lax.dot_general(lb, w, (((1,), (1,)), ((), ())),
                         preferred_element_type=jnp.float32)
    n2 = jnp.sum(cb * cb, axis=1, keepdims=True)
    norm = jnp.sqrt(n2)
    cb = cb / jnp.clip(norm, 1e-12) * math.sqrt(D)
    cb_ref[...] = cb
    cbn_ref[...] = jnp.sum(cb * cb, axis=1, keepdims=True)


def _make_codebook(latent_basis, W):
    return pl.pallas_call(
        _codebook_body,
        out_shape=[
            jax.ShapeDtypeStruct((K, D), jnp.float32),
            jax.ShapeDtypeStruct((K, 1), jnp.float32),
        ],
    )(latent_basis, W)


def _argmin_body(z_ref, cb_ref, cbn_ref, idx_ref, loss_ref):
    z = z_ref[...]
    cb = cb_ref[...]
    mm = lax.dot_general(z, cb, (((1,), (1,)), ((), ())),
                         preferred_element_type=jnp.float32)
    zn = jnp.sum(z * z, axis=1, keepdims=True)
    d = zn - 2.0 * mm + cbn_ref[...]
    dmin = jnp.min(d, axis=1, keepdims=True)
    iota = lax.broadcasted_iota(jnp.int32, d.shape, 1)
    idx = jnp.min(jnp.where(d == dmin, iota, jnp.int32(K)), axis=1)
    idx_ref[0, 0, :] = idx

    @pl.when(pl.program_id(0) == 0)
    def _():
        loss_ref[...] = jnp.zeros((1, 1), jnp.float32)
    loss_ref[...] += jnp.sum(dmin, keepdims=True).reshape(1, 1)


def _argmin_distances(z_flat, codebook, cbn_row):
    return pl.pallas_call(
        _argmin_body,
        grid=(GRID_M,),
        in_specs=[
            pl.BlockSpec((BM, D), lambda i: (i, 0)),
            pl.BlockSpec((K, D), lambda i: (0, 0)),
            pl.BlockSpec((1, K), lambda i: (0, 0)),
        ],
        out_specs=[
            pl.BlockSpec((1, 1, BM), lambda i: (i, 0, 0)),
            pl.BlockSpec((1, 1), lambda i: (0, 0)),
        ],
        out_shape=[
            jax.ShapeDtypeStruct((GRID_M, 1, BM), jnp.int32),
            jax.ShapeDtypeStruct((1, 1), jnp.float32),
        ],
        compiler_params=pltpu.CompilerParams(
            dimension_semantics=("arbitrary",)),
    )(z_flat, codebook, cbn_row)


def _sc_gather_body(cb_hbm, idx_hbm, zeros_hbm, ones_hbm,
                    zq_hbm, counts_hbm,
                    idx_v, rows_v, ones_v, hist_v, hist_sh, sem):
    cid = lax.axis_index("c")
    sid = lax.axis_index("s")
    wid = cid * NS + sid

    pltpu.sync_copy(idx_hbm.at[pl.ds(wid * NCHUNK, NCHUNK)], idx_v)
    pltpu.sync_copy(ones_hbm, ones_v)

    @pl.when(sid == 0)
    def _():
        pltpu.sync_copy(zeros_hbm, hist_sh)
    plsc.subcore_barrier()

    for j in range(NCHUNK):
        pltpu.sync_copy(ones_v, hist_sh.at[idx_v.at[j]], add=True)

    base = wid * TOK_PER_W
    for j in range(NCHUNK):
        pltpu.async_copy(cb_hbm.at[idx_v.at[j]], rows_v, sem).wait()
        pltpu.sync_copy(rows_v, zq_hbm.at[pl.ds(base + j * CHUNK, CHUNK)])

    plsc.subcore_barrier()

    @pl.when(sid == 0)
    def _():
        pltpu.sync_copy(hist_sh, hist_v)
        pltpu.sync_copy(hist_v, counts_hbm.at[cid])


def _sc_gather(codebook, idx2d, zeros_i32, ones_i32):
    mesh = plsc.VectorSubcoreMesh(core_axis_name="c", subcore_axis_name="s")
    kfn = pl.kernel(
        _sc_gather_body,
        out_type=[
            jax.ShapeDtypeStruct((NTOK, D), jnp.float32),
            jax.ShapeDtypeStruct((NC, K), jnp.int32),
        ],
        mesh=mesh,
        scratch_types=[
            pltpu.VMEM((NCHUNK, CHUNK), jnp.int32),
            pltpu.VMEM((CHUNK, D), jnp.float32),
            pltpu.VMEM((CHUNK,), jnp.int32),
            pltpu.VMEM((K,), jnp.int32),
            pltpu.VMEM_SHARED((K,), jnp.int32),
            pltpu.SemaphoreType.DMA,
        ],
    )
    return kfn(codebook, idx2d, zeros_i32, ones_i32)


def _finalize_body(counts_ref, losssum_ref, loss_ref, perp_ref, usage_ref):
    c = counts_ref[...]
    total = c[0:1, :] + c[1:2, :]
    avg = total.astype(jnp.float32) / float(NTOK)
    ent = jnp.sum(avg * jnp.log(avg + 1e-10), keepdims=True).reshape(1, 1)
    perp_ref[...] = jnp.exp(-ent)
    used = jnp.sum((total > 0).astype(jnp.float32), keepdims=True)
    usage_ref[...] = used.reshape(1, 1) / float(K)
    mse = losssum_ref[...] / float(NTOK * D)
    loss_ref[...] = mse + COMMIT * mse


def _finalize(counts, loss_sum):
    return pl.pallas_call(
        _finalize_body,
        out_shape=[
            jax.ShapeDtypeStruct((1, 1), jnp.float32),
            jax.ShapeDtypeStruct((1, 1), jnp.float32),
            jax.ShapeDtypeStruct((1, 1), jnp.float32),
        ],
    )(counts, loss_sum)


@jax.jit
def kernel(z_e, latent_basis, W):
    Bb, Tt, Dd = z_e.shape
    z_flat = z_e.reshape(-1, Dd)

    codebook, cbn_col = _make_codebook(latent_basis, W)
    cbn_row = cbn_col.reshape(1, K)

    idx_blocks, loss_sum = _argmin_distances(z_flat, codebook, cbn_row)
    indices_flat = idx_blocks.reshape(-1)

    idx2d = indices_flat.reshape(NTOK // CHUNK, CHUNK)
    zeros_i32 = jnp.zeros((K,), jnp.int32)
    ones_i32 = jnp.ones((CHUNK,), jnp.int32)
    z_q_flat, counts = _sc_gather(codebook, idx2d, zeros_i32, ones_i32)

    vq_loss, perplexity, usage = _finalize(counts, loss_sum)

    z_q = z_q_flat.reshape(Bb, Tt, Dd)
    indices = indices_flat.reshape(Bb, Tt)
    return (z_q, vq_loss[0, 0], indices, perplexity[0, 0], usage[0, 0])
